# baseline (speedup 1.0000x reference)
import jax
import jax.numpy as jnp
from jax.experimental import pallas as pl

_YBLK = 40
_BBLK = 2


def _bcast_body(row_ref, col_ref, out_ref):
    nf = row_ref.shape[1]
    re = row_ref[...]
    ce = col_ref[...]
    yb, w = out_ref.shape[1], out_ref.shape[2]
    nb = out_ref.shape[0]
    out_ref[:, :, :, nf:] = jnp.broadcast_to(re[None, :, None, :], (nb, yb, w, nf))
    out_ref[:, :, :, :nf] = jnp.broadcast_to(ce[None, None, :, :], (nb, yb, w, nf))


def kernel(mask, row_embed, col_embed):
    batch = mask.shape[0]
    h, w = mask.shape[-2], mask.shape[-1]
    nf = row_embed.shape[1]

    grid = (batch // _BBLK, h // _YBLK)
    p = pl.pallas_call(
        _bcast_body,
        grid=grid,
        in_specs=[
            pl.BlockSpec((_YBLK, nf), lambda b, i: (i, 0)),
            pl.BlockSpec((w, nf), lambda b, i: (0, 0)),
        ],
        out_specs=pl.BlockSpec((_BBLK, _YBLK, w, 2 * nf), lambda b, i: (b, i, 0, 0)),
        out_shape=jax.ShapeDtypeStruct((batch, h, w, 2 * nf), row_embed.dtype),
    )(row_embed, col_embed)
    return jnp.transpose(p, (0, 3, 1, 2))


# speedup vs baseline: 1.0178x; 1.0178x over previous
import jax
import jax.numpy as jnp
from jax.experimental import pallas as pl

_YBLK = 40
_BBLK = 1


def _bcast_body(row_ref, col_ref, out_ref):
    nf = row_ref.shape[1]
    re = row_ref[...]
    ce = col_ref[...]
    yb, w = out_ref.shape[1], out_ref.shape[2]
    nb = out_ref.shape[0]
    out_ref[:, :, :, nf:] = jnp.broadcast_to(re[None, :, None, :], (nb, yb, w, nf))
    out_ref[:, :, :, :nf] = jnp.broadcast_to(ce[None, None, :, :], (nb, yb, w, nf))


def kernel(mask, row_embed, col_embed):
    batch = mask.shape[0]
    h, w = mask.shape[-2], mask.shape[-1]
    nf = row_embed.shape[1]

    grid = (batch // _BBLK, h // _YBLK)
    p = pl.pallas_call(
        _bcast_body,
        grid=grid,
        in_specs=[
            pl.BlockSpec((_YBLK, nf), lambda b, i: (i, 0)),
            pl.BlockSpec((w, nf), lambda b, i: (0, 0)),
        ],
        out_specs=pl.BlockSpec((_BBLK, _YBLK, w, 2 * nf), lambda b, i: (b, i, 0, 0)),
        out_shape=jax.ShapeDtypeStruct((batch, h, w, 2 * nf), row_embed.dtype),
    )(row_embed, col_embed)
    return jnp.transpose(p, (0, 3, 1, 2))
